# baseline (speedup 1.0000x reference)
import functools

import jax
import jax.numpy as jnp
from jax import lax
from jax.experimental import pallas as pl
from jax.experimental.pallas import tpu as pltpu
from jax.experimental.pallas import tpu_sc as plsc

N_X = 1000000
N_Y = 1000000
N_E = 64
B = 16384

NC = 2
NS = 16
L = 16
NW = NC * NS
BPW = B // NW
CHUNK = 128
NCHUNK = BPW // CHUNK
GPC = CHUNK // L
WIDE = 2 * N_E

RB = 512
NBLK = -(-N_X // RB)

_EPS2 = 1e-24


def _rsqrt(v):
    i = lax.bitcast_convert_type(v, jnp.int32)
    i = jnp.int32(0x5F3759DF) - lax.shift_right_logical(i, 1)
    y = lax.bitcast_convert_type(i, jnp.float32)
    for _ in range(3):
        y = y * (1.5 - 0.5 * v * y * y)
    return y


def _tc_relayout_kernel(xt_ref, o_ref):
    eye2 = jnp.tile(jnp.eye(N_E, dtype=jnp.float32), (1, 2))
    o_ref[...] = jax.lax.dot_general(
        xt_ref[...], eye2,
        dimension_numbers=(((0,), (0,)), ((), ())),
        preferred_element_type=jnp.float32)


def _tc_relayout(xt):
    n = xt.shape[1]
    return pl.pallas_call(
        _tc_relayout_kernel,
        grid=(-(-n // RB),),
        in_specs=[pl.BlockSpec((N_E, RB), lambda i: (0, i))],
        out_specs=pl.BlockSpec((RB, WIDE), lambda i: (i, 0)),
        out_shape=jax.ShapeDtypeStruct((n, WIDE), jnp.float32),
    )(xt)


def _sc_kernel(x_hbm, y_hbm, xt_hbm, yt_hbm, out_hbm,
               xi_v, yi_v, xb0, xb1, yb0, yb1, out_v, sem0, sem1):
    wid = lax.axis_index("s") * NC + lax.axis_index("c")
    base = wid * BPW

    pltpu.sync_copy(x_hbm.at[pl.ds(base, BPW)], xi_v)
    pltpu.sync_copy(y_hbm.at[pl.ds(base, BPW)], yi_v)

    xbufs = (xb0, xb1)
    ybufs = (yb0, yb1)
    sems = (sem0, sem1)

    def fire(c):
        s = pl.ds(c * CHUNK, CHUNK)
        b = c % 2
        pltpu.async_copy(xt_hbm.at[xi_v.at[s]], xbufs[b], sems[b])
        pltpu.async_copy(yt_hbm.at[yi_v.at[s]], ybufs[b], sems[b])

    def drain(b):
        pltpu.make_async_copy(
            xt_hbm.at[pl.ds(0, CHUNK)], xbufs[b], sems[b]).wait()
        pltpu.make_async_copy(
            yt_hbm.at[pl.ds(0, CHUNK)], ybufs[b], sems[b]).wait()

    lanes = lax.iota(jnp.int32, L)

    fire(0)
    for c in range(NCHUNK):
        b = c % 2
        if c + 1 < NCHUNK:
            fire(c + 1)
        drain(b)
        xb = xbufs[b]
        yb = ybufs[b]

        def group_body(g, _, xb=xb, yb=yb, c=c):
            rows = g * L + lanes
            zero = jnp.zeros((L,), jnp.float32)
            axy0, axy1 = zero, zero
            axx0, axx1 = zero, zero
            ayy0, ayy1 = zero, zero
            for j in range(N_E):
                col = jnp.full((L,), j, jnp.int32)
                vx = plsc.load_gather(xb, [rows, col])
                vy = plsc.load_gather(yb, [rows, col])
                if j % 2 == 0:
                    axy0 = axy0 + vx * vy
                    axx0 = axx0 + vx * vx
                    ayy0 = ayy0 + vy * vy
                else:
                    axy1 = axy1 + vx * vy
                    axx1 = axx1 + vx * vx
                    ayy1 = ayy1 + vy * vy
            axy = axy0 + axy1
            axx = axx0 + axx1
            ayy = ayy0 + ayy1
            res = (axy * _rsqrt(jnp.maximum(axx, _EPS2))
                       * _rsqrt(jnp.maximum(ayy, _EPS2)))
            plsc.store_scatter(out_v, [c * CHUNK + rows], res)
            return 0

        lax.fori_loop(0, GPC, group_body, 0)

    pltpu.sync_copy(out_v, out_hbm.at[pl.ds(base, BPW)])


@jax.jit
def _run(x, y, x_table, y_table):
    xd = _tc_relayout(x_table.T)
    yd = _tc_relayout(y_table.T)
    mesh = plsc.VectorSubcoreMesh(core_axis_name="c", subcore_axis_name="s")
    f = functools.partial(
        pl.kernel,
        mesh=mesh,
        out_type=jax.ShapeDtypeStruct((B,), jnp.float32),
        scratch_types=[
            pltpu.VMEM((BPW,), jnp.int32),
            pltpu.VMEM((BPW,), jnp.int32),
            pltpu.VMEM((CHUNK, WIDE), jnp.float32),
            pltpu.VMEM((CHUNK, WIDE), jnp.float32),
            pltpu.VMEM((CHUNK, WIDE), jnp.float32),
            pltpu.VMEM((CHUNK, WIDE), jnp.float32),
            pltpu.VMEM((BPW,), jnp.float32),
            pltpu.SemaphoreType.DMA,
            pltpu.SemaphoreType.DMA,
        ],
        compiler_params=pltpu.CompilerParams(
            needs_layout_passes=False, use_tc_tiling_on_sc=True),
    )(_sc_kernel)
    return f(x, y, xd, yd)


def kernel(x, y, x_table, y_table):
    return _run(x.astype(jnp.int32), y.astype(jnp.int32), x_table, y_table)


# speedup vs baseline: 4.4807x; 4.4807x over previous
import functools

import jax
import jax.numpy as jnp
from jax import lax
from jax.experimental import pallas as pl
from jax.experimental.pallas import tpu as pltpu
from jax.experimental.pallas import tpu_sc as plsc

N_X = 1000000
N_Y = 1000000
N_E = 64
B = 16384

NC = 2
NS = 16
L = 16
NW = NC * NS
BPW = B // NW
CHUNK = 128
NCHUNK = BPW // CHUNK
GPC = CHUNK // L
WIDE = 2 * N_E

RB = 8192
NBLK = -(-N_X // RB)

_EPS2 = 1e-24


def _rsqrt(v):
    i = lax.bitcast_convert_type(v, jnp.int32)
    i = jnp.int32(0x5F3759DF) - lax.shift_right_logical(i, 1)
    y = lax.bitcast_convert_type(i, jnp.float32)
    for _ in range(3):
        y = y * (1.5 - 0.5 * v * y * y)
    return y


def _tc_relayout_kernel(xt_ref, o_ref):
    eye2 = jnp.tile(jnp.eye(N_E, dtype=jnp.float32), (1, 2))
    o_ref[...] = jax.lax.dot_general(
        xt_ref[...], eye2,
        dimension_numbers=(((0,), (0,)), ((), ())),
        preferred_element_type=jnp.float32)


def _tc_relayout(xt):
    n = xt.shape[1]
    return pl.pallas_call(
        _tc_relayout_kernel,
        grid=(-(-n // RB),),
        in_specs=[pl.BlockSpec((N_E, RB), lambda i: (0, i))],
        out_specs=pl.BlockSpec((RB, WIDE), lambda i: (i, 0)),
        out_shape=jax.ShapeDtypeStruct((n, WIDE), jnp.float32),
    )(xt)


def _sc_kernel(x_hbm, y_hbm, xt_hbm, yt_hbm, out_hbm,
               xi_v, yi_v, xb0, xb1, yb0, yb1, out_v, sem0, sem1):
    wid = lax.axis_index("s") * NC + lax.axis_index("c")
    base = wid * BPW

    pltpu.sync_copy(x_hbm.at[pl.ds(base, BPW)], xi_v)
    pltpu.sync_copy(y_hbm.at[pl.ds(base, BPW)], yi_v)

    xbufs = (xb0, xb1)
    ybufs = (yb0, yb1)
    sems = (sem0, sem1)

    def fire(c):
        s = pl.ds(c * CHUNK, CHUNK)
        b = c % 2
        pltpu.async_copy(xt_hbm.at[xi_v.at[s]], xbufs[b], sems[b])
        pltpu.async_copy(yt_hbm.at[yi_v.at[s]], ybufs[b], sems[b])

    def drain(b):
        pltpu.make_async_copy(
            xt_hbm.at[pl.ds(0, CHUNK)], xbufs[b], sems[b]).wait()
        pltpu.make_async_copy(
            yt_hbm.at[pl.ds(0, CHUNK)], ybufs[b], sems[b]).wait()

    lanes = lax.iota(jnp.int32, L)

    fire(0)
    for c in range(NCHUNK):
        b = c % 2
        if c + 1 < NCHUNK:
            fire(c + 1)
        drain(b)
        xb = xbufs[b]
        yb = ybufs[b]

        def group_body(g, _, xb=xb, yb=yb, c=c):
            rows = g * L + lanes
            zero = jnp.zeros((L,), jnp.float32)
            axy0, axy1 = zero, zero
            axx0, axx1 = zero, zero
            ayy0, ayy1 = zero, zero
            for j in range(N_E):
                col = jnp.full((L,), j, jnp.int32)
                vx = plsc.load_gather(xb, [rows, col])
                vy = plsc.load_gather(yb, [rows, col])
                if j % 2 == 0:
                    axy0 = axy0 + vx * vy
                    axx0 = axx0 + vx * vx
                    ayy0 = ayy0 + vy * vy
                else:
                    axy1 = axy1 + vx * vy
                    axx1 = axx1 + vx * vx
                    ayy1 = ayy1 + vy * vy
            axy = axy0 + axy1
            axx = axx0 + axx1
            ayy = ayy0 + ayy1
            res = (axy * _rsqrt(jnp.maximum(axx, _EPS2))
                       * _rsqrt(jnp.maximum(ayy, _EPS2)))
            plsc.store_scatter(out_v, [c * CHUNK + rows], res)
            return 0

        lax.fori_loop(0, GPC, group_body, 0)

    pltpu.sync_copy(out_v, out_hbm.at[pl.ds(base, BPW)])


@jax.jit
def _run(x, y, x_table, y_table):
    xd = _tc_relayout(x_table.T)
    yd = _tc_relayout(y_table.T)
    mesh = plsc.VectorSubcoreMesh(core_axis_name="c", subcore_axis_name="s")
    f = functools.partial(
        pl.kernel,
        mesh=mesh,
        out_type=jax.ShapeDtypeStruct((B,), jnp.float32),
        scratch_types=[
            pltpu.VMEM((BPW,), jnp.int32),
            pltpu.VMEM((BPW,), jnp.int32),
            pltpu.VMEM((CHUNK, WIDE), jnp.float32),
            pltpu.VMEM((CHUNK, WIDE), jnp.float32),
            pltpu.VMEM((CHUNK, WIDE), jnp.float32),
            pltpu.VMEM((CHUNK, WIDE), jnp.float32),
            pltpu.VMEM((BPW,), jnp.float32),
            pltpu.SemaphoreType.DMA,
            pltpu.SemaphoreType.DMA,
        ],
        compiler_params=pltpu.CompilerParams(
            needs_layout_passes=False, use_tc_tiling_on_sc=True),
    )(_sc_kernel)
    return f(x, y, xd, yd)


def kernel(x, y, x_table, y_table):
    return _run(x.astype(jnp.int32), y.astype(jnp.int32), x_table, y_table)


# speedup vs baseline: 5.3411x; 1.1920x over previous
import functools

import jax
import jax.numpy as jnp
from jax import lax
from jax.experimental import pallas as pl
from jax.experimental.pallas import tpu as pltpu
from jax.experimental.pallas import tpu_sc as plsc

N_X = 1000000
N_Y = 1000000
N_E = 64
B = 16384

NC = 2
NS = 16
L = 16
NW = NC * NS
BPW = B // NW
CHUNK = 128
NCHUNK = BPW // CHUNK
GPC = CHUNK // L
WIDE = 2 * N_E

RB = 8192
NHB = 61
HSPLIT = NHB * RB
TAIL = 2 * HSPLIT
NFAT = (NHB + 1) * RB

_EPS2 = 1e-24


def _rsqrt(v):
    i = lax.bitcast_convert_type(v, jnp.int32)
    i = jnp.int32(0x5F3759DF) - lax.shift_right_logical(i, 1)
    y = lax.bitcast_convert_type(i, jnp.float32)
    for _ in range(3):
        y = y * (1.5 - 0.5 * v * y * y)
    return y


def _tc_relayout_kernel(xta_ref, xtb_ref, o_ref):
    eye = jnp.eye(N_E, dtype=jnp.float32)
    dn = (((0,), (0,)), ((), ()))
    ta = jax.lax.dot_general(xta_ref[...], eye, dimension_numbers=dn,
                             preferred_element_type=jnp.float32)
    tb = jax.lax.dot_general(xtb_ref[...], eye, dimension_numbers=dn,
                             preferred_element_type=jnp.float32)
    o_ref[...] = jnp.concatenate([ta, tb], axis=1)


def _tc_relayout(xt):
    nblk_tail = TAIL // RB
    return pl.pallas_call(
        _tc_relayout_kernel,
        grid=(NHB + 1,),
        in_specs=[
            pl.BlockSpec((N_E, RB),
                         lambda i: (0, jnp.where(i < NHB, i, nblk_tail))),
            pl.BlockSpec((N_E, RB),
                         lambda i: (0, jnp.where(i < NHB, NHB + i, nblk_tail))),
        ],
        out_specs=pl.BlockSpec((RB, WIDE), lambda i: (i, 0)),
        out_shape=jax.ShapeDtypeStruct((NFAT, WIDE), jnp.float32),
    )(xt, xt)


def _sc_kernel(x_hbm, y_hbm, xt_hbm, yt_hbm, out_hbm,
               xi_v, yi_v, xf_v, yf_v, xb0, xb1, yb0, yb1, out_v,
               sem0, sem1):
    wid = lax.axis_index("s") * NC + lax.axis_index("c")
    base = wid * BPW

    pltpu.sync_copy(x_hbm.at[pl.ds(base, BPW)], xi_v)
    pltpu.sync_copy(y_hbm.at[pl.ds(base, BPW)], yi_v)

    for k in range(BPW // L):
        s = pl.ds(k * L, L)
        rx = xi_v[s]
        ry = yi_v[s]
        xf_v[s] = rx - jnp.where(rx >= HSPLIT, HSPLIT, 0)
        yf_v[s] = ry - jnp.where(ry >= HSPLIT, HSPLIT, 0)

    xbufs = (xb0, xb1)
    ybufs = (yb0, yb1)
    sems = (sem0, sem1)

    def fire(c):
        s = pl.ds(c * CHUNK, CHUNK)
        b = c % 2
        pltpu.async_copy(xt_hbm.at[xf_v.at[s]], xbufs[b], sems[b])
        pltpu.async_copy(yt_hbm.at[yf_v.at[s]], ybufs[b], sems[b])

    def drain(b):
        pltpu.make_async_copy(
            xt_hbm.at[pl.ds(0, CHUNK)], xbufs[b], sems[b]).wait()
        pltpu.make_async_copy(
            yt_hbm.at[pl.ds(0, CHUNK)], ybufs[b], sems[b]).wait()

    lanes = lax.iota(jnp.int32, L)

    fire(0)
    for c in range(NCHUNK):
        b = c % 2
        if c + 1 < NCHUNK:
            fire(c + 1)
        drain(b)
        xb = xbufs[b]
        yb = ybufs[b]

        def group_body(g, _, xb=xb, yb=yb, c=c):
            rows = g * L + lanes
            rx = plsc.load_gather(xi_v, [c * CHUNK + rows])
            ry = plsc.load_gather(yi_v, [c * CHUNK + rows])
            xoff = jnp.where((rx >= HSPLIT) & (rx < TAIL), N_E, 0)
            yoff = jnp.where((ry >= HSPLIT) & (ry < TAIL), N_E, 0)
            zero = jnp.zeros((L,), jnp.float32)
            axy0, axy1 = zero, zero
            axx0, axx1 = zero, zero
            ayy0, ayy1 = zero, zero
            for j in range(N_E):
                vx = plsc.load_gather(xb, [rows, xoff + j])
                vy = plsc.load_gather(yb, [rows, yoff + j])
                if j % 2 == 0:
                    axy0 = axy0 + vx * vy
                    axx0 = axx0 + vx * vx
                    ayy0 = ayy0 + vy * vy
                else:
                    axy1 = axy1 + vx * vy
                    axx1 = axx1 + vx * vx
                    ayy1 = ayy1 + vy * vy
            axy = axy0 + axy1
            axx = axx0 + axx1
            ayy = ayy0 + ayy1
            res = (axy * _rsqrt(jnp.maximum(axx, _EPS2))
                       * _rsqrt(jnp.maximum(ayy, _EPS2)))
            plsc.store_scatter(out_v, [c * CHUNK + rows], res)
            return 0

        lax.fori_loop(0, GPC, group_body, 0)

    pltpu.sync_copy(out_v, out_hbm.at[pl.ds(base, BPW)])


@jax.jit
def _run(x, y, x_table, y_table):
    xd = _tc_relayout(x_table.T)
    yd = _tc_relayout(y_table.T)
    mesh = plsc.VectorSubcoreMesh(core_axis_name="c", subcore_axis_name="s")
    f = functools.partial(
        pl.kernel,
        mesh=mesh,
        out_type=jax.ShapeDtypeStruct((B,), jnp.float32),
        scratch_types=[
            pltpu.VMEM((BPW,), jnp.int32),
            pltpu.VMEM((BPW,), jnp.int32),
            pltpu.VMEM((BPW,), jnp.int32),
            pltpu.VMEM((BPW,), jnp.int32),
            pltpu.VMEM((CHUNK, WIDE), jnp.float32),
            pltpu.VMEM((CHUNK, WIDE), jnp.float32),
            pltpu.VMEM((CHUNK, WIDE), jnp.float32),
            pltpu.VMEM((CHUNK, WIDE), jnp.float32),
            pltpu.VMEM((BPW,), jnp.float32),
            pltpu.SemaphoreType.DMA,
            pltpu.SemaphoreType.DMA,
        ],
        compiler_params=pltpu.CompilerParams(
            needs_layout_passes=False, use_tc_tiling_on_sc=True),
    )(_sc_kernel)
    return f(x, y, xd, yd)


def kernel(x, y, x_table, y_table):
    return _run(x.astype(jnp.int32), y.astype(jnp.int32), x_table, y_table)


# speedup vs baseline: 5.5058x; 1.0308x over previous
import functools

import jax
import jax.numpy as jnp
from jax import lax
from jax.experimental import pallas as pl
from jax.experimental.pallas import tpu as pltpu
from jax.experimental.pallas import tpu_sc as plsc

N_X = 1000000
N_Y = 1000000
N_E = 64
B = 16384

NC = 2
NS = 16
L = 16
NW = NC * NS
BPW = B // NW
CHUNK = 128
NCHUNK = BPW // CHUNK
GPC = CHUNK // L
WIDE = 2 * N_E

RB = 16384
NHB = 30
HSPLIT = NHB * RB
TAIL = 2 * HSPLIT
NTB = 2
NFAT = (NHB + NTB) * RB

_EPS2 = 1e-24


def _rsqrt(v):
    i = lax.bitcast_convert_type(v, jnp.int32)
    i = jnp.int32(0x5F3759DF) - lax.shift_right_logical(i, 1)
    y = lax.bitcast_convert_type(i, jnp.float32)
    for _ in range(3):
        y = y * (1.5 - 0.5 * v * y * y)
    return y


def _tc_relayout_kernel(xta_ref, xtb_ref, o_ref):
    eye = jnp.eye(N_E, dtype=jnp.float32)
    dn = (((0,), (0,)), ((), ()))
    ta = jax.lax.dot_general(xta_ref[...], eye, dimension_numbers=dn,
                             preferred_element_type=jnp.float32)
    tb = jax.lax.dot_general(xtb_ref[...], eye, dimension_numbers=dn,
                             preferred_element_type=jnp.float32)
    o_ref[...] = jnp.concatenate([ta, tb], axis=1)


def _tc_relayout(xt):
    return pl.pallas_call(
        _tc_relayout_kernel,
        grid=(NHB + NTB,),
        in_specs=[
            pl.BlockSpec((N_E, RB),
                         lambda i: (0, jnp.where(i < NHB, i, i + NHB))),
            pl.BlockSpec((N_E, RB), lambda i: (0, i + NHB)),
        ],
        out_specs=pl.BlockSpec((RB, WIDE), lambda i: (i, 0)),
        out_shape=jax.ShapeDtypeStruct((NFAT, WIDE), jnp.float32),
    )(xt, xt)


def _sc_kernel(x_hbm, y_hbm, xt_hbm, yt_hbm, out_hbm,
               xi_v, yi_v, xf_v, yf_v, xb0, xb1, yb0, yb1, out_v,
               sem0, sem1):
    wid = lax.axis_index("s") * NC + lax.axis_index("c")
    base = wid * BPW

    pltpu.sync_copy(x_hbm.at[pl.ds(base, BPW)], xi_v)
    pltpu.sync_copy(y_hbm.at[pl.ds(base, BPW)], yi_v)

    for k in range(BPW // L):
        s = pl.ds(k * L, L)
        rx = xi_v[s]
        ry = yi_v[s]
        xf_v[s] = rx - jnp.where(rx >= HSPLIT, HSPLIT, 0)
        yf_v[s] = ry - jnp.where(ry >= HSPLIT, HSPLIT, 0)

    xbufs = (xb0, xb1)
    ybufs = (yb0, yb1)
    sems = (sem0, sem1)

    def fire(c):
        s = pl.ds(c * CHUNK, CHUNK)
        b = c % 2
        pltpu.async_copy(xt_hbm.at[xf_v.at[s]], xbufs[b], sems[b])
        pltpu.async_copy(yt_hbm.at[yf_v.at[s]], ybufs[b], sems[b])

    def drain(b):
        pltpu.make_async_copy(
            xt_hbm.at[pl.ds(0, CHUNK)], xbufs[b], sems[b]).wait()
        pltpu.make_async_copy(
            yt_hbm.at[pl.ds(0, CHUNK)], ybufs[b], sems[b]).wait()

    lanes = lax.iota(jnp.int32, L)

    fire(0)
    for c in range(NCHUNK):
        b = c % 2
        if c + 1 < NCHUNK:
            fire(c + 1)
        drain(b)
        xb = xbufs[b]
        yb = ybufs[b]

        def group_body(g, _, xb=xb, yb=yb, c=c):
            rows = g * L + lanes
            rx = plsc.load_gather(xi_v, [c * CHUNK + rows])
            ry = plsc.load_gather(yi_v, [c * CHUNK + rows])
            xoff = jnp.where((rx >= HSPLIT) & (rx < TAIL), N_E, 0)
            yoff = jnp.where((ry >= HSPLIT) & (ry < TAIL), N_E, 0)
            zero = jnp.zeros((L,), jnp.float32)
            axy0, axy1 = zero, zero
            axx0, axx1 = zero, zero
            ayy0, ayy1 = zero, zero
            for j in range(N_E):
                vx = plsc.load_gather(xb, [rows, xoff + j])
                vy = plsc.load_gather(yb, [rows, yoff + j])
                if j % 2 == 0:
                    axy0 = axy0 + vx * vy
                    axx0 = axx0 + vx * vx
                    ayy0 = ayy0 + vy * vy
                else:
                    axy1 = axy1 + vx * vy
                    axx1 = axx1 + vx * vx
                    ayy1 = ayy1 + vy * vy
            axy = axy0 + axy1
            axx = axx0 + axx1
            ayy = ayy0 + ayy1
            res = (axy * _rsqrt(jnp.maximum(axx, _EPS2))
                       * _rsqrt(jnp.maximum(ayy, _EPS2)))
            plsc.store_scatter(out_v, [c * CHUNK + rows], res)
            return 0

        lax.fori_loop(0, GPC, group_body, 0)

    pltpu.sync_copy(out_v, out_hbm.at[pl.ds(base, BPW)])


@jax.jit
def _run(x, y, x_table, y_table):
    xd = _tc_relayout(x_table.T)
    yd = _tc_relayout(y_table.T)
    mesh = plsc.VectorSubcoreMesh(core_axis_name="c", subcore_axis_name="s")
    f = functools.partial(
        pl.kernel,
        mesh=mesh,
        out_type=jax.ShapeDtypeStruct((B,), jnp.float32),
        scratch_types=[
            pltpu.VMEM((BPW,), jnp.int32),
            pltpu.VMEM((BPW,), jnp.int32),
            pltpu.VMEM((BPW,), jnp.int32),
            pltpu.VMEM((BPW,), jnp.int32),
            pltpu.VMEM((CHUNK, WIDE), jnp.float32),
            pltpu.VMEM((CHUNK, WIDE), jnp.float32),
            pltpu.VMEM((CHUNK, WIDE), jnp.float32),
            pltpu.VMEM((CHUNK, WIDE), jnp.float32),
            pltpu.VMEM((BPW,), jnp.float32),
            pltpu.SemaphoreType.DMA,
            pltpu.SemaphoreType.DMA,
        ],
        compiler_params=pltpu.CompilerParams(
            needs_layout_passes=False, use_tc_tiling_on_sc=True),
    )(_sc_kernel)
    return f(x, y, xd, yd)


def kernel(x, y, x_table, y_table):
    return _run(x.astype(jnp.int32), y.astype(jnp.int32), x_table, y_table)
